# Initial kernel scaffold; baseline (speedup 1.0000x reference)
import jax
import jax.numpy as jnp
from jax.experimental import pallas as pl


def kernel(x, edge_index, x0, W_in, W_skip, conv_W, conv_b, W_fc, b_fc):
    raise NotImplementedError("write your pallas kernel here")


# speedup vs baseline: 3.6274x; 3.6274x over previous
import jax
import jax.numpy as jnp
from jax import lax
from jax.experimental import pallas as pl
from jax.experimental.pallas import tpu as pltpu
from jax.experimental.pallas import tpu_sc as plsc

N = 10000
D = 128
E = 320000
NC = 2
NS = 16
NW = NC * NS
KB = 128
EPW = -(-E // (NW * KB)) * KB
E_PAD = EPW * NW
NBLK = EPW // KB
NPAD = N + 16
NROWS = 10112
RPW = NROWS // NS
BN = 2000
_MESH = plsc.VectorSubcoreMesh(core_axis_name="c", subcore_axis_name="s")


def _sc_scatter_body(table, gidx, sidx, zeros2d, out, idx_g, idx_s, rows, acc, sem):
    c = lax.axis_index("c")
    s = lax.axis_index("s")
    base = (c * NS + s) * EPW
    pltpu.sync_copy(zeros2d.at[pl.ds(s * RPW, RPW)], acc.at[pl.ds(s * RPW, RPW)])
    plsc.subcore_barrier()

    def body(j, carry):
        off = base + j * KB
        pltpu.sync_copy(gidx.at[pl.ds(off, KB)], idx_g)
        pltpu.sync_copy(sidx.at[pl.ds(off, KB)], idx_s)
        pltpu.async_copy(table.at[idx_g], rows, sem).wait()
        pltpu.sync_copy(rows, acc.at[idx_s], add=True)
        return carry

    lax.fori_loop(0, NBLK, body, 0)
    plsc.subcore_barrier()
    pltpu.sync_copy(acc.at[pl.ds(s * RPW, RPW)], out.at[c, pl.ds(s * RPW, RPW)])


_sc_scatter = pl.kernel(
    _sc_scatter_body,
    out_type=jax.ShapeDtypeStruct((NC, NROWS, D), jnp.float32),
    mesh=_MESH,
    scratch_types=[
        pltpu.VMEM((KB,), jnp.int32),
        pltpu.VMEM((KB,), jnp.int32),
        pltpu.VMEM((KB, D), jnp.float32),
        pltpu.VMEM_SHARED((NROWS, D), jnp.float32),
        pltpu.SemaphoreType.DMA,
    ],
    compiler_params=pltpu.CompilerParams(needs_layout_passes=False),
)


def _sc_gamma_body(table, s_pad, gidx, sidx, zeros2d, zeros1d,
                   t_out, scat_out, deg_out,
                   idx_g, idx_s, rows, s_loc, scat_loc, deg_loc, acc, sem):
    c = lax.axis_index("c")
    s = lax.axis_index("s")
    w = c * NS + s
    base = w * EPW
    pltpu.sync_copy(zeros2d.at[pl.ds(s * RPW, RPW)], acc.at[pl.ds(s * RPW, RPW)])
    pltpu.sync_copy(s_pad, s_loc)
    pltpu.sync_copy(zeros1d, scat_loc)
    pltpu.sync_copy(zeros1d, deg_loc)
    plsc.subcore_barrier()
    ones = jnp.ones((16,), jnp.float32)

    def body(j, carry):
        off = base + j * KB
        pltpu.sync_copy(gidx.at[pl.ds(off, KB)], idx_g)
        pltpu.sync_copy(sidx.at[pl.ds(off, KB)], idx_s)
        pltpu.async_copy(table.at[idx_g], rows, sem).wait()
        pltpu.sync_copy(rows, acc.at[idx_s], add=True)
        for e in range(KB // 16):
            iv_g = idx_g[pl.ds(e * 16, 16)]
            iv_s = idx_s[pl.ds(e * 16, 16)]
            sv = plsc.load_gather(s_loc, [iv_g])
            plsc.addupdate_scatter(scat_loc, [iv_s], sv)
            plsc.addupdate_scatter(deg_loc, [iv_s], ones)
        return carry

    lax.fori_loop(0, NBLK, body, 0)
    plsc.subcore_barrier()
    pltpu.sync_copy(acc.at[pl.ds(s * RPW, RPW)], t_out.at[c, pl.ds(s * RPW, RPW)])
    pltpu.sync_copy(scat_loc, scat_out.at[w])
    pltpu.sync_copy(deg_loc, deg_out.at[w])


_sc_gamma = pl.kernel(
    _sc_gamma_body,
    out_type=(
        jax.ShapeDtypeStruct((NC, NROWS, D), jnp.float32),
        jax.ShapeDtypeStruct((NW, NPAD), jnp.float32),
        jax.ShapeDtypeStruct((NW, NPAD), jnp.float32),
    ),
    mesh=_MESH,
    scratch_types=[
        pltpu.VMEM((KB,), jnp.int32),
        pltpu.VMEM((KB,), jnp.int32),
        pltpu.VMEM((KB, D), jnp.float32),
        pltpu.VMEM((NPAD,), jnp.float32),
        pltpu.VMEM((NPAD,), jnp.float32),
        pltpu.VMEM((NPAD,), jnp.float32),
        pltpu.VMEM_SHARED((NROWS, D), jnp.float32),
        pltpu.SemaphoreType.DMA,
    ],
    compiler_params=pltpu.CompilerParams(needs_layout_passes=False),
)


_G = N // BN
_row = pl.BlockSpec((BN, D), lambda i: (i, 0))
_col1 = pl.BlockSpec((BN, 1), lambda i: (i, 0))
_wmat = pl.BlockSpec((D, D), lambda i: (0, 0))
_brow = pl.BlockSpec((1, D), lambda i: (0, 0))
_parts = pl.BlockSpec((NW, BN), lambda i: (0, i))


def _t1_body(x_ref, x0_ref, win_ref, w0_ref, wskip_ref, hw0_ref, xs_ref):
    h = jnp.dot(x_ref[...], win_ref[...], preferred_element_type=jnp.float32)
    hw0_ref[...] = jnp.dot(h, w0_ref[...], preferred_element_type=jnp.float32)
    h0 = jnp.dot(x0_ref[...], win_ref[...], preferred_element_type=jnp.float32)
    xs_ref[...] = jnp.dot(h0, wskip_ref[...], preferred_element_type=jnp.float32)


_t1 = pl.pallas_call(
    _t1_body,
    grid=(_G,),
    in_specs=[_row, _row, _wmat, _wmat, _wmat],
    out_specs=[_row, _row],
    out_shape=[jax.ShapeDtypeStruct((N, D), jnp.float32)] * 2,
)


def _t2_body(a0_ref, a1_ref, b_ref, w_ref, xa_ref, hw_ref):
    xa = jnp.maximum(a0_ref[...] + a1_ref[...] + b_ref[...], 0.0)
    xa_ref[...] = xa
    hw_ref[...] = jnp.dot(xa, w_ref[...], preferred_element_type=jnp.float32)


_t2 = pl.pallas_call(
    _t2_body,
    grid=(_G,),
    in_specs=[_row, _row, _brow, _wmat],
    out_specs=[_row, _row],
    out_shape=[jax.ShapeDtypeStruct((N, D), jnp.float32)] * 2,
)


def _t3_body(a0_ref, a1_ref, b_ref, xa_ref, s_ref):
    xa = jnp.maximum(a0_ref[...] + a1_ref[...] + b_ref[...], 0.0)
    xa_ref[...] = xa
    s_ref[...] = jnp.sum(xa * xa, axis=1, keepdims=True)


_t3 = pl.pallas_call(
    _t3_body,
    grid=(_G,),
    in_specs=[_row, _row, _brow],
    out_specs=[_row, _col1],
    out_shape=[
        jax.ShapeDtypeStruct((N, D), jnp.float32),
        jax.ShapeDtypeStruct((N, 1), jnp.float32),
    ],
)


def _red_body(scat_ref, deg_ref, scatv_ref, degv_ref):
    scatv_ref[...] = jnp.sum(scat_ref[...], axis=0)[:N, None]
    degv_ref[...] = jnp.sum(deg_ref[...], axis=0)[:N, None]


_red = pl.pallas_call(
    _red_body,
    out_shape=[jax.ShapeDtypeStruct((N, 1), jnp.float32)] * 2,
)


def _gate_body(hp_ref, xa_ref, xs_ref, s_ref, t0_ref, t1_ref, scat_ref,
               deg_ref, sq_ref, w_ref, b_ref, h_ref, mm_ref):
    xa = xa_ref[...]
    t = t0_ref[...] + t1_ref[...]
    dot = jnp.sum(xa * t, axis=1, keepdims=True)
    scat = scat_ref[...]
    deg = deg_ref[...]
    num = deg * s_ref[...] + scat - 2.0 * dot
    gs = jnp.tanh(num / (deg + 1e-10))
    sq = sq_ref[...]
    h_new = (hp_ref[...] + gs * xa + sq * xs_ref[...]) / (1.0 + gs + sq)
    h_ref[...] = h_new
    mm_ref[...] = jnp.dot(h_new, w_ref[...], preferred_element_type=jnp.float32) + b_ref[...]


_gate = pl.pallas_call(
    _gate_body,
    grid=(_G,),
    in_specs=[_row, _row, _row, _col1, _row, _row, _col1, _col1, _col1,
              _wmat, _brow],
    out_specs=[_row, _row],
    out_shape=[jax.ShapeDtypeStruct((N, D), jnp.float32)] * 2,
)


def kernel(x, edge_index, x0, W_in, W_skip, conv_W, conv_b, W_fc, b_fc):
    src = edge_index[0].astype(jnp.int32)
    dst = edge_index[1].astype(jnp.int32)
    pad = E_PAD - E
    zi = jnp.zeros((pad,), jnp.int32)
    di = jnp.full((pad,), N, jnp.int32)
    g_agg = jnp.concatenate([src, zi])
    s_agg = jnp.concatenate([dst, di])
    g_gam = jnp.concatenate([dst, zi])
    s_gam = jnp.concatenate([src, di])
    zeros2d = jnp.zeros((NROWS, D), jnp.float32)
    zeros1d = jnp.zeros((NPAD,), jnp.float32)
    sq1 = 0.5 + 0.4 * jax.random.uniform(
        jax.random.fold_in(jax.random.key(42), 1), (N, 1), dtype=jnp.float32)
    sq2 = 0.5 + 0.4 * jax.random.uniform(
        jax.random.fold_in(jax.random.key(42), 2), (N, 1), dtype=jnp.float32)
    zb = jnp.zeros((1, D), jnp.float32)

    hw0, xs = _t1(x, x0, W_in, conv_W[0], W_skip)
    aggp = _sc_scatter(hw0, g_agg, s_agg, zeros2d)
    x_agg0, hw1 = _t2(aggp[0], aggp[1], conv_b[0][None], conv_W[1])
    aggp = _sc_scatter(hw1, g_agg, s_agg, zeros2d)
    x_agg1, s1 = _t3(aggp[0], aggp[1], conv_b[1][None])
    tp, scat1, deg1 = _sc_gamma(x_agg1, jnp.pad(s1[:, 0], (0, NPAD - N)),
                                g_gam, s_gam, zeros2d, zeros1d)
    scatv, degv = _red(scat1, deg1)
    h2, hw2 = _gate(x_agg0, x_agg1, xs, s1, tp[0], tp[1], scatv,
                    degv, sq1, conv_W[2], zb)
    aggp = _sc_scatter(hw2, g_agg, s_agg, zeros2d)
    x_agg2, s2 = _t3(aggp[0], aggp[1], conv_b[2][None])
    tp, scat2, deg2 = _sc_gamma(x_agg2, jnp.pad(s2[:, 0], (0, NPAD - N)),
                                g_gam, s_gam, zeros2d, zeros1d)
    scatv, degv = _red(scat2, deg2)
    _, out = _gate(h2, x_agg2, xs, s2, tp[0], tp[1], scatv,
                   degv, sq2, W_fc, b_fc[None])
    return out
